# Initial kernel scaffold; baseline (speedup 1.0000x reference)
import jax
import jax.numpy as jnp
from jax.experimental import pallas as pl


def kernel(pos, mapping, atom_types, x_0, k_const):
    raise NotImplementedError("write your pallas kernel here")


# speedup vs baseline: 172.8520x; 172.8520x over previous
import functools

import jax
import jax.numpy as jnp
from jax import lax
from jax.experimental import pallas as pl
from jax.experimental.pallas import tpu as pltpu
from jax.experimental.pallas import tpu_sc as plsc

N_ATOMS = 10000
N_BONDS = 320000
N_TYPES = 20

_NC = 2
_NS = 16
_NW = _NC * _NS
_L = 16
_E_PER = N_BONDS // _NW
_CHUNKS = _E_PER // _L


def _sqrt16(s):
    i = lax.bitcast_convert_type(s, jnp.int32)
    i = jnp.int32(0x5F3759DF) - lax.shift_right_logical(i, 1)
    r = lax.bitcast_convert_type(i, jnp.float32)
    half = s * jnp.float32(0.5)
    for _ in range(3):
        r = r * (jnp.float32(1.5) - half * r * r)
    return s * r


def _body(posx_h, posy_h, posz_h, typ_h, x0_h, k_h, src_h, dst_h, out_h,
          posx_v, posy_v, posz_v, typ_v, x0_v, k_v, src_v, dst_v, out_v):
    wid = lax.axis_index("s") * _NC + lax.axis_index("c")
    base = wid * _E_PER

    pltpu.sync_copy(posx_h, posx_v)
    pltpu.sync_copy(posy_h, posy_v)
    pltpu.sync_copy(posz_h, posz_v)
    pltpu.sync_copy(typ_h, typ_v)
    pltpu.sync_copy(x0_h, x0_v)
    pltpu.sync_copy(k_h, k_v)
    pltpu.sync_copy(src_h.at[pl.ds(base, _E_PER)], src_v)
    pltpu.sync_copy(dst_h.at[pl.ds(base, _E_PER)], dst_v)

    def chunk(c, carry):
        off = c * _L
        i = src_v[pl.ds(off, _L)]
        j = dst_v[pl.ds(off, _L)]
        xi = plsc.load_gather(posx_v, [i])
        yi = plsc.load_gather(posy_v, [i])
        zi = plsc.load_gather(posz_v, [i])
        xj = plsc.load_gather(posx_v, [j])
        yj = plsc.load_gather(posy_v, [j])
        zj = plsc.load_gather(posz_v, [j])
        ti = plsc.load_gather(typ_v, [i])
        tj = plsc.load_gather(typ_v, [j])
        t = ti * N_TYPES + tj
        x0e = plsc.load_gather(x0_v, [t])
        ke = plsc.load_gather(k_v, [t])
        dx = xi - xj
        dy = yi - yj
        dz = zi - zj
        s = dx * dx + dy * dy + dz * dz + jnp.float32(1e-12)
        d = _sqrt16(s)
        diff = d - x0e
        out_v[pl.ds(off, _L)] = ke * diff * diff
        return carry

    lax.fori_loop(0, _CHUNKS, chunk, 0)
    pltpu.sync_copy(out_v, out_h.at[pl.ds(base, _E_PER)])


@functools.partial(
    pl.kernel,
    mesh=plsc.VectorSubcoreMesh(core_axis_name="c", subcore_axis_name="s"),
    out_type=jax.ShapeDtypeStruct((N_BONDS,), jnp.float32),
    compiler_params=pltpu.CompilerParams(needs_layout_passes=False),
    scratch_types=[
        pltpu.VMEM((N_ATOMS,), jnp.float32),
        pltpu.VMEM((N_ATOMS,), jnp.float32),
        pltpu.VMEM((N_ATOMS,), jnp.float32),
        pltpu.VMEM((N_ATOMS,), jnp.int32),
        pltpu.VMEM((N_TYPES * N_TYPES,), jnp.float32),
        pltpu.VMEM((N_TYPES * N_TYPES,), jnp.float32),
        pltpu.VMEM((_E_PER,), jnp.int32),
        pltpu.VMEM((_E_PER,), jnp.int32),
        pltpu.VMEM((_E_PER,), jnp.float32),
    ],
)
def _harmonic_sc(posx, posy, posz, typ, x0f, kf, src, dst, out,
                 posx_v, posy_v, posz_v, typ_v, x0_v, k_v, src_v, dst_v, out_v):
    _body(posx, posy, posz, typ, x0f, kf, src, dst, out,
          posx_v, posy_v, posz_v, typ_v, x0_v, k_v, src_v, dst_v, out_v)


def kernel(pos, mapping, atom_types, x_0, k_const):
    pos = pos.astype(jnp.float32)
    posx = pos[:, 0]
    posy = pos[:, 1]
    posz = pos[:, 2]
    typ = atom_types.astype(jnp.int32)
    src = mapping[0].astype(jnp.int32)
    dst = mapping[1].astype(jnp.int32)
    x0f = x_0.reshape(-1).astype(jnp.float32)
    kf = k_const.reshape(-1).astype(jnp.float32)
    return _harmonic_sc(posx, posy, posz, typ, x0f, kf, src, dst)
